# baseline (device time: 16210 ns/iter reference)
import jax
import jax.numpy as jnp
from jax import lax
from jax.experimental import pallas as pl
from jax.experimental.pallas import tpu as pltpu

N_DEV = 4
H_GLOBAL = 512


def kernel(x, Wp):
    b, h_per, w, c = x.shape
    c_out = Wp.shape[1]
    n_norm = float(H_GLOBAL * w)

    xt = jnp.transpose(x, (0, 1, 3, 2))

    def body(x_ref, wp_ref, out_ref, local_ref, stats_ref, tot_ref,
             send_sems, recv_sems, exit_sem):
        my = lax.axis_index("i")

        barrier_sem = pltpu.get_barrier_semaphore()
        for off in (1, 2, 3):
            pl.semaphore_signal(
                barrier_sem, inc=1,
                device_id=((my + off) % N_DEV,),
                device_id_type=pl.DeviceIdType.MESH,
            )

        xv = x_ref[...]
        ps = jnp.sum(xv, axis=(1, 3))
        pss = jnp.sum(xv * xv, axis=(1, 3))
        local_ref[...] = jnp.concatenate([ps, pss], axis=0)

        pl.semaphore_wait(barrier_sem, N_DEV - 1)

        rdmas = []
        for off in (1, 2, 3):
            rdma = pltpu.make_async_remote_copy(
                src_ref=local_ref,
                dst_ref=stats_ref.at[off - 1],
                send_sem=send_sems.at[off - 1],
                recv_sem=recv_sems.at[off - 1],
                device_id=((my + off) % N_DEV,),
                device_id_type=pl.DeviceIdType.MESH,
            )
            rdma.start()
            rdmas.append(rdma)
        for rdma in rdmas:
            rdma.wait_recv()
        for rdma in rdmas:
            rdma.wait_send()

        tot_ref[...] = (
            local_ref[...] + stats_ref[0] + stats_ref[1] + stats_ref[2]
        )

        for off in (1, 2, 3):
            pl.semaphore_signal(
                exit_sem, inc=1,
                device_id=((my + off) % N_DEV,),
                device_id_type=pl.DeviceIdType.MESH,
            )

        tot = tot_ref[...]
        mean = tot[:b, :] / n_norm
        var = tot[b:, :] / n_norm - mean * mean
        inv = lax.rsqrt(var + 1e-5)
        mb = mean.astype(jnp.bfloat16)[:, None, :, None]
        ib = inv.astype(jnp.bfloat16)[:, None, :, None]

        wb = wp_ref[...].astype(jnp.bfloat16)
        n_chunks = 4
        ch = h_per // n_chunks
        for k in range(n_chunks):
            xk = xv[:, k * ch:(k + 1) * ch]
            hn = (xk.astype(jnp.bfloat16) - mb) * ib
            a2 = hn * jax.nn.sigmoid(hn)
            o = lax.dot_general(
                a2, wb,
                dimension_numbers=(((2,), (0,)), ((), ())),
                preferred_element_type=jnp.float32,
            )
            out_ref[:, k * ch:(k + 1) * ch] = o.astype(jnp.bfloat16)

        pl.semaphore_wait(exit_sem, N_DEV - 1)

    return pl.pallas_call(
        body,
        out_shape=jax.ShapeDtypeStruct((b, h_per, w, c_out), jnp.bfloat16),
        in_specs=[
            pl.BlockSpec(memory_space=pltpu.VMEM),
            pl.BlockSpec(memory_space=pltpu.VMEM),
        ],
        out_specs=pl.BlockSpec(memory_space=pltpu.VMEM),
        scratch_shapes=[
            pltpu.VMEM((2 * b, c), jnp.float32),
            pltpu.VMEM((N_DEV - 1, 2 * b, c), jnp.float32),
            pltpu.VMEM((2 * b, c), jnp.float32),
            pltpu.SemaphoreType.DMA((N_DEV - 1,)),
            pltpu.SemaphoreType.DMA((N_DEV - 1,)),
            pltpu.SemaphoreType.REGULAR,
        ],
        compiler_params=pltpu.CompilerParams(collective_id=0),
    )(xt, Wp)
